# Initial kernel scaffold; baseline (speedup 1.0000x reference)
import jax
import jax.numpy as jnp
from jax.experimental import pallas as pl


def kernel(feat, score, centers):
    raise NotImplementedError("write your pallas kernel here")


# speedup vs baseline: 1.3309x; 1.3309x over previous
import jax
import jax.numpy as jnp
from jax import lax
from jax.experimental import pallas as pl
from jax.experimental.pallas import tpu as pltpu

_FEAT_DIM = 128
_ALPHA = 0.001
_BETA = 0.5
_EPS = 1e-8
_T = 8192
_HALF = 32
_K = 6


def _loss1_body(nor_ref, cen_ref, out_ref, acc_ref):
    i = pl.program_id(0)

    @pl.when(i == 0)
    def _init():
        acc_ref[0, 0] = jnp.float32(0.0)

    x = nor_ref[0]
    c = cen_ref[0]
    cn = jnp.sqrt(jnp.sum(c * c))
    dots = jnp.sum(x * c[None, :], axis=1)
    n2 = jnp.sum(x * x, axis=1)
    den = jnp.maximum(jnp.sqrt(n2) * cn, _EPS)
    acc_ref[0, 0] += jnp.sum(dots / den)

    @pl.when(i == pl.num_programs(0) - 1)
    def _fin():
        mean_cos = acc_ref[0, 0] / jnp.float32(_HALF * _T)
        out_ref[0, 0] = _ALPHA * (1.0 - mean_cos)


def _triplet_body(sc_ref, cen_ref, feat_hbm, out_ref,
                  gat_ref, idxv_ref, idxs_ref, sem_idx, sem_g):
    s = sc_ref[...]
    iota = lax.broadcasted_iota(jnp.int32, (_HALF, _T), 1)
    ibig = jnp.int32(2**30)
    inf = jnp.float32(jnp.inf)

    cols = []
    cur = s
    for _ in range(_K):
        m = jnp.min(cur, axis=1, keepdims=True)
        idx = jnp.min(jnp.where(cur == m, iota, ibig), axis=1, keepdims=True)
        cur = jnp.where(iota == idx, inf, cur)
        cols.append(idx)
    cur = s
    for _ in range(_K):
        m = jnp.max(cur, axis=1, keepdims=True)
        idx = jnp.min(jnp.where(cur == m, iota, ibig), axis=1, keepdims=True)
        cur = jnp.where(iota == idx, -inf, cur)
        cols.append(idx)
    idxv_ref[...] = jnp.concatenate(cols, axis=1)

    cp = pltpu.make_async_copy(idxv_ref, idxs_ref, sem_idx)
    cp.start()
    cp.wait()

    def _issue(r, carry):
        v = r // 12
        k = r - v * 12
        row = idxs_ref[v, k]
        pltpu.make_async_copy(
            feat_hbm.at[_HALF + v, pl.ds(row, 1), :],
            gat_ref.at[v, pl.ds(k, 1), :],
            sem_g,
        ).start()
        return carry

    lax.fori_loop(0, _HALF * 12, _issue, 0)
    pltpu.make_async_copy(
        feat_hbm.at[pl.ds(0, _HALF), pl.ds(0, 12), :], gat_ref, sem_g
    ).wait()

    g = gat_ref[...]
    c = cen_ref[0]
    cn = jnp.sqrt(jnp.sum(c * c))
    dots = jnp.sum(g * c[None, None, :], axis=2)
    n2 = jnp.sum(g * g, axis=2)
    cos = dots / jnp.maximum(jnp.sqrt(n2) * cn, _EPS)
    dpos = (1.0 - cos[:, :_K]) * 0.5
    dneg = (1.0 - cos[:, _K:]) * 0.5
    l2 = jnp.mean(jnp.maximum(dpos - dneg + 1.0, 0.0))
    out_ref[0, 0] = _BETA * l2


def kernel(feat, score, centers):
    score2 = score.reshape(2 * _HALF, _T)
    cen2 = centers.reshape(1, _FEAT_DIM)

    l1 = pl.pallas_call(
        _loss1_body,
        grid=(_HALF,),
        in_specs=[
            pl.BlockSpec((1, _T, _FEAT_DIM), lambda i: (i, 0, 0)),
            pl.BlockSpec((1, _FEAT_DIM), lambda i: (0, 0)),
        ],
        out_specs=pl.BlockSpec((1, 1), lambda i: (0, 0),
                               memory_space=pltpu.SMEM),
        out_shape=jax.ShapeDtypeStruct((1, 1), jnp.float32),
        scratch_shapes=[pltpu.SMEM((1, 1), jnp.float32)],
    )(feat, cen2)

    l2 = pl.pallas_call(
        _triplet_body,
        grid=(1,),
        in_specs=[
            pl.BlockSpec((_HALF, _T), lambda i: (1, 0)),
            pl.BlockSpec((1, _FEAT_DIM), lambda i: (0, 0)),
            pl.BlockSpec(memory_space=pl.ANY),
        ],
        out_specs=pl.BlockSpec((1, 1), lambda i: (0, 0),
                               memory_space=pltpu.SMEM),
        out_shape=jax.ShapeDtypeStruct((1, 1), jnp.float32),
        scratch_shapes=[
            pltpu.VMEM((_HALF, 12, _FEAT_DIM), jnp.float32),
            pltpu.VMEM((_HALF, 12), jnp.int32),
            pltpu.SMEM((_HALF, 12), jnp.int32),
            pltpu.SemaphoreType.DMA,
            pltpu.SemaphoreType.DMA,
        ],
    )(score2, cen2, feat)

    return l1[0, 0] + l2[0, 0]


# speedup vs baseline: 1.6965x; 1.2747x over previous
import jax
import jax.numpy as jnp
from jax import lax
from jax.experimental import pallas as pl
from jax.experimental.pallas import tpu as pltpu

_FEAT_DIM = 128
_ALPHA = 0.001
_BETA = 0.5
_EPS = 1e-8
_T = 8192
_HALF = 32
_K = 6


def _loss1_body(nor_ref, cen_ref, out_ref, acc_ref):
    i = pl.program_id(0)

    @pl.when(i == 0)
    def _init():
        acc_ref[0, 0] = jnp.float32(0.0)

    x = nor_ref[0]
    c = cen_ref[0]
    cinv = lax.rsqrt(jnp.sum(c * c))
    dots = jnp.sum(x * c[None, :], axis=1)
    n2 = jnp.sum(x * x, axis=1)
    r = jnp.minimum(lax.rsqrt(n2) * cinv, jnp.float32(1.0 / _EPS))
    acc_ref[0, 0] += jnp.sum(dots * r)

    @pl.when(i == pl.num_programs(0) - 1)
    def _fin():
        mean_cos = acc_ref[0, 0] / jnp.float32(_HALF * _T)
        out_ref[0, 0] = _ALPHA * (1.0 - mean_cos)


def _triplet_body(sc_ref, cen_ref, feat_hbm, out_ref,
                  gat_ref, idxv_ref, idxs_ref, sem_idx, sem_g):
    s = sc_ref[...]
    iota = lax.broadcasted_iota(jnp.int32, (_HALF, _T), 1)
    ibig = jnp.int32(2**30)
    inf = jnp.float32(jnp.inf)

    cols = []
    cur = s
    for _ in range(_K):
        m = jnp.min(cur, axis=1, keepdims=True)
        idx = jnp.min(jnp.where(cur == m, iota, ibig), axis=1, keepdims=True)
        cur = jnp.where(iota == idx, inf, cur)
        cols.append(idx)
    cur = s
    for _ in range(_K):
        m = jnp.max(cur, axis=1, keepdims=True)
        idx = jnp.min(jnp.where(cur == m, iota, ibig), axis=1, keepdims=True)
        cur = jnp.where(iota == idx, -inf, cur)
        cols.append(idx)
    idxv_ref[...] = jnp.concatenate(cols, axis=1)

    cp = pltpu.make_async_copy(idxv_ref, idxs_ref, sem_idx)
    cp.start()
    cp.wait()

    def _issue(r, carry):
        v = r // 12
        k = r - v * 12
        row = idxs_ref[v, k]
        pltpu.make_async_copy(
            feat_hbm.at[_HALF + v, pl.ds(row, 1), :],
            gat_ref.at[v, pl.ds(k, 1), :],
            sem_g,
        ).start()
        return carry

    lax.fori_loop(0, _HALF * 12, _issue, 0)
    pltpu.make_async_copy(
        feat_hbm.at[pl.ds(0, _HALF), pl.ds(0, 12), :], gat_ref, sem_g
    ).wait()

    g = gat_ref[...]
    c = cen_ref[0]
    cn = jnp.sqrt(jnp.sum(c * c))
    dots = jnp.sum(g * c[None, None, :], axis=2)
    n2 = jnp.sum(g * g, axis=2)
    cos = dots / jnp.maximum(jnp.sqrt(n2) * cn, _EPS)
    dpos = (1.0 - cos[:, :_K]) * 0.5
    dneg = (1.0 - cos[:, _K:]) * 0.5
    l2 = jnp.mean(jnp.maximum(dpos - dneg + 1.0, 0.0))
    out_ref[0, 0] = _BETA * l2


def kernel(feat, score, centers):
    score2 = score.reshape(2 * _HALF, _T)
    cen2 = centers.reshape(1, _FEAT_DIM)

    l1 = pl.pallas_call(
        _loss1_body,
        grid=(_HALF,),
        in_specs=[
            pl.BlockSpec((1, _T, _FEAT_DIM), lambda i: (i, 0, 0)),
            pl.BlockSpec((1, _FEAT_DIM), lambda i: (0, 0)),
        ],
        out_specs=pl.BlockSpec((1, 1), lambda i: (0, 0),
                               memory_space=pltpu.SMEM),
        out_shape=jax.ShapeDtypeStruct((1, 1), jnp.float32),
        scratch_shapes=[pltpu.SMEM((1, 1), jnp.float32)],
    )(feat, cen2)

    l2 = pl.pallas_call(
        _triplet_body,
        grid=(1,),
        in_specs=[
            pl.BlockSpec((_HALF, _T), lambda i: (1, 0)),
            pl.BlockSpec((1, _FEAT_DIM), lambda i: (0, 0)),
            pl.BlockSpec(memory_space=pl.ANY),
        ],
        out_specs=pl.BlockSpec((1, 1), lambda i: (0, 0),
                               memory_space=pltpu.SMEM),
        out_shape=jax.ShapeDtypeStruct((1, 1), jnp.float32),
        scratch_shapes=[
            pltpu.VMEM((_HALF, 12, _FEAT_DIM), jnp.float32),
            pltpu.VMEM((_HALF, 12), jnp.int32),
            pltpu.SMEM((_HALF, 12), jnp.int32),
            pltpu.SemaphoreType.DMA,
            pltpu.SemaphoreType.DMA,
        ],
    )(score2, cen2, feat)

    return l1[0, 0] + l2[0, 0]


# speedup vs baseline: 2.0191x; 1.1901x over previous
import jax
import jax.numpy as jnp
from jax import lax
from jax.experimental import pallas as pl
from jax.experimental.pallas import tpu as pltpu

_FEAT_DIM = 128
_ALPHA = 0.001
_BETA = 0.5
_EPS = 1e-8
_T = 8192
_HALF = 32
_K = 6


def _loss1_body(nor_ref, cen_ref, out_ref, acc_ref):
    i = pl.program_id(0)

    @pl.when(i == 0)
    def _init():
        acc_ref[0, 0] = jnp.float32(0.0)

    x = nor_ref[0]
    c = cen_ref[0]
    cinv = lax.rsqrt(jnp.sum(c * c))
    xb = x.astype(jnp.bfloat16)
    xcat = jnp.concatenate([xb, xb * xb], axis=1)
    cb = c.astype(jnp.bfloat16)
    z = jnp.zeros((_FEAT_DIM,), jnp.bfloat16)
    o = jnp.ones((_FEAT_DIM,), jnp.bfloat16)
    w = jnp.stack([jnp.concatenate([cb, z]), jnp.concatenate([z, o])], axis=1)
    p = lax.dot_general(xcat, w, (((1,), (0,)), ((), ())),
                        preferred_element_type=jnp.float32)
    pt = p.T
    dots = pt[0:1, :]
    n2 = pt[1:2, :]
    r = jnp.minimum(lax.rsqrt(n2) * cinv, jnp.float32(1.0 / _EPS))
    acc_ref[0, 0] += jnp.sum(dots * r)

    @pl.when(i == pl.num_programs(0) - 1)
    def _fin():
        mean_cos = acc_ref[0, 0] / jnp.float32(_HALF * _T)
        out_ref[0, 0] = _ALPHA * (1.0 - mean_cos)


def _triplet_body(sc_ref, cen_ref, feat_hbm, out_ref,
                  gat_ref, idxv_ref, idxs_ref, sem_idx, sem_g):
    s = sc_ref[...]
    iota = lax.broadcasted_iota(jnp.int32, (_HALF, _T), 1)
    ibig = jnp.int32(2**30)
    inf = jnp.float32(jnp.inf)

    cols = []
    cur = s
    for _ in range(_K):
        m = jnp.min(cur, axis=1, keepdims=True)
        idx = jnp.min(jnp.where(cur == m, iota, ibig), axis=1, keepdims=True)
        cur = jnp.where(iota == idx, inf, cur)
        cols.append(idx)
    cur = s
    for _ in range(_K):
        m = jnp.max(cur, axis=1, keepdims=True)
        idx = jnp.min(jnp.where(cur == m, iota, ibig), axis=1, keepdims=True)
        cur = jnp.where(iota == idx, -inf, cur)
        cols.append(idx)
    idxv_ref[...] = jnp.concatenate(cols, axis=1)

    cp = pltpu.make_async_copy(idxv_ref, idxs_ref, sem_idx)
    cp.start()
    cp.wait()

    def _issue(r, carry):
        v = r // 12
        k = r - v * 12
        row = idxs_ref[v, k]
        pltpu.make_async_copy(
            feat_hbm.at[_HALF + v, pl.ds(row, 1), :],
            gat_ref.at[v, pl.ds(k, 1), :],
            sem_g,
        ).start()
        return carry

    lax.fori_loop(0, _HALF * 12, _issue, 0)
    pltpu.make_async_copy(
        feat_hbm.at[pl.ds(0, _HALF), pl.ds(0, 12), :], gat_ref, sem_g
    ).wait()

    g = gat_ref[...]
    c = cen_ref[0]
    cn = jnp.sqrt(jnp.sum(c * c))
    dots = jnp.sum(g * c[None, None, :], axis=2)
    n2 = jnp.sum(g * g, axis=2)
    cos = dots / jnp.maximum(jnp.sqrt(n2) * cn, _EPS)
    dpos = (1.0 - cos[:, :_K]) * 0.5
    dneg = (1.0 - cos[:, _K:]) * 0.5
    l2 = jnp.mean(jnp.maximum(dpos - dneg + 1.0, 0.0))
    out_ref[0, 0] = _BETA * l2


def kernel(feat, score, centers):
    score2 = score.reshape(2 * _HALF, _T)
    cen2 = centers.reshape(1, _FEAT_DIM)

    l1 = pl.pallas_call(
        _loss1_body,
        grid=(_HALF,),
        in_specs=[
            pl.BlockSpec((1, _T, _FEAT_DIM), lambda i: (i, 0, 0)),
            pl.BlockSpec((1, _FEAT_DIM), lambda i: (0, 0)),
        ],
        out_specs=pl.BlockSpec((1, 1), lambda i: (0, 0),
                               memory_space=pltpu.SMEM),
        out_shape=jax.ShapeDtypeStruct((1, 1), jnp.float32),
        scratch_shapes=[pltpu.SMEM((1, 1), jnp.float32)],
    )(feat, cen2)

    l2 = pl.pallas_call(
        _triplet_body,
        grid=(1,),
        in_specs=[
            pl.BlockSpec((_HALF, _T), lambda i: (1, 0)),
            pl.BlockSpec((1, _FEAT_DIM), lambda i: (0, 0)),
            pl.BlockSpec(memory_space=pl.ANY),
        ],
        out_specs=pl.BlockSpec((1, 1), lambda i: (0, 0),
                               memory_space=pltpu.SMEM),
        out_shape=jax.ShapeDtypeStruct((1, 1), jnp.float32),
        scratch_shapes=[
            pltpu.VMEM((_HALF, 12, _FEAT_DIM), jnp.float32),
            pltpu.VMEM((_HALF, 12), jnp.int32),
            pltpu.SMEM((_HALF, 12), jnp.int32),
            pltpu.SemaphoreType.DMA,
            pltpu.SemaphoreType.DMA,
        ],
    )(score2, cen2, feat)

    return l1[0, 0] + l2[0, 0]
